# baseline (speedup 1.0000x reference)
import functools

import jax
import jax.numpy as jnp
from jax import lax
from jax.experimental import pallas as pl
from jax.experimental.pallas import tpu as pltpu
from jax.experimental.pallas import tpu_sc as plsc

_VOCAB = 100000
_DIM = 300
_HEAD = 256
_TAIL = _DIM - _HEAD

_NC = 2
_NS = 16
_NW = _NC * _NS
_BATCH = 16384
_CHUNK = 128
_CPW = _BATCH // (_NW * _CHUNK)
_BPW = _BATCH // _NW


def _embed_body(idx_hbm, head_hbm, tail_hbm, out_hbm,
                idx_v, bufs0, bufs1, sem0, sem1):
    wid = lax.axis_index("s") * _NC + lax.axis_index("c")
    pltpu.sync_copy(idx_hbm.at[pl.ds(wid * _BPW, _BPW)], idx_v)
    bufs = (bufs0, bufs1)
    sems = (sem0, sem1)

    def start(j):
        b = j % 2
        ii = idx_v.at[pl.ds(j * _CHUNK, _CHUNK)]
        return (
            pltpu.async_copy(head_hbm.at[ii], bufs[b][0], sems[b]),
            pltpu.async_copy(tail_hbm.at[ii], bufs[b][1], sems[b]),
        )

    copies = [start(0), None]
    for j in range(_CPW):
        b = j % 2
        if j + 1 < _CPW:
            copies[(j + 1) % 2] = start(j + 1)
        for cp in copies[b]:
            cp.wait()
        rows = pl.ds((wid * _CPW + j) * _CHUNK, _CHUNK)
        pltpu.sync_copy(bufs[b][0], out_hbm.at[rows, pl.ds(0, _HEAD)])
        pltpu.sync_copy(bufs[b][1], out_hbm.at[rows, pl.ds(_HEAD, 128)])


_embed_lookup = functools.partial(
    pl.kernel,
    out_type=jax.ShapeDtypeStruct((_BATCH, 384), jnp.float32),
    mesh=plsc.VectorSubcoreMesh(core_axis_name="c", subcore_axis_name="s"),
    scratch_types=[
        pltpu.VMEM((_BPW,), jnp.int32),
        (pltpu.VMEM((_CHUNK, _HEAD), jnp.float32),
         pltpu.VMEM((_CHUNK, 128), jnp.float32)),
        (pltpu.VMEM((_CHUNK, _HEAD), jnp.float32),
         pltpu.VMEM((_CHUNK, 128), jnp.float32)),
        pltpu.SemaphoreType.DMA,
        pltpu.SemaphoreType.DMA,
    ],
)(_embed_body)


def kernel(tokens, table):
    idx = tokens.astype(jnp.int32)
    head = lax.slice(table.T, (0, 0), (_HEAD, _VOCAB)).T
    tail = lax.pad(table, jnp.float32(0),
                   [(0, 0, 0), (-_HEAD, 128 - _TAIL, 0)])
    out_pad = _embed_lookup(idx, head, tail)
    return out_pad[:, :_DIM]
